# baseline (speedup 1.0000x reference)
import jax
import jax.numpy as jnp
from jax.experimental import pallas as pl

_GRID = 25


def _copy_body(xs_ref, xt_ref, ea_ref, u_ref, oxs_ref, oxt_ref, oea_ref, ou_ref):
    oxs_ref[...] = xs_ref[...]
    oxt_ref[...] = xt_ref[...]
    oea_ref[...] = ea_ref[...]

    @pl.when(pl.program_id(0) == 0)
    def _():
        ou_ref[...] = u_ref[...]


def kernel(x_s, x_t, edge_index, edge_attr, u, batch_e, batch_s, batch_t):
    del edge_index, batch_e, batch_s, batch_t
    n_s, d_feat = x_s.shape
    e, d_edge = edge_attr.shape
    ea2 = edge_attr.reshape(e * d_edge // 128, 128)

    bx = n_s // _GRID
    be = ea2.shape[0] // _GRID

    xs_o, xt_o, ea_o, u_o = pl.pallas_call(
        _copy_body,
        grid=(_GRID,),
        in_specs=[
            pl.BlockSpec((bx, d_feat), lambda i: (i, 0)),
            pl.BlockSpec((bx, d_feat), lambda i: (i, 0)),
            pl.BlockSpec((be, 128), lambda i: (i, 0)),
            pl.BlockSpec(u.shape, lambda i: (0, 0)),
        ],
        out_specs=[
            pl.BlockSpec((bx, d_feat), lambda i: (i, 0)),
            pl.BlockSpec((bx, d_feat), lambda i: (i, 0)),
            pl.BlockSpec((be, 128), lambda i: (i, 0)),
            pl.BlockSpec(u.shape, lambda i: (0, 0)),
        ],
        out_shape=[
            jax.ShapeDtypeStruct(x_s.shape, x_s.dtype),
            jax.ShapeDtypeStruct(x_t.shape, x_t.dtype),
            jax.ShapeDtypeStruct(ea2.shape, ea2.dtype),
            jax.ShapeDtypeStruct(u.shape, u.dtype),
        ],
    )(x_s, x_t, ea2, u)

    return (xs_o, xt_o, ea_o.reshape(e, d_edge), u_o)
